# baseline (device time: 10783 ns/iter reference)
import functools

import jax
import jax.numpy as jnp
from jax import lax
from jax.experimental import pallas as pl
from jax.experimental.pallas import tpu as pltpu

K = 8


def kernel(x, dy, gamma):
    m, d = x.shape
    mb = m // K

    def body(x_ref, dy_ref, out_ref, acc_ref, comm_ref, send_sem, recv_sem):
        i = pl.program_id(0)
        my_x = lax.axis_index("x")
        my_y = lax.axis_index("y")
        my_z = lax.axis_index("z")
        nbr = (my_x, 1 - my_y, my_z)
        barrier_sem = pltpu.get_barrier_semaphore()

        @pl.when(i == 0)
        def _():
            pl.semaphore_signal(
                barrier_sem, inc=1,
                device_id=nbr, device_id_type=pl.DeviceIdType.MESH,
            )

        xv = x_ref[:, :]
        dyv = dy_ref[:, :]
        mu = jnp.mean(xv, axis=1, keepdims=True)
        msq = jnp.mean(xv * xv, axis=1, keepdims=True)
        rstd = lax.rsqrt(msq - mu * mu + 1e-5)
        xhat = (xv - mu) * rstd
        dgamma_c = jnp.sum(dyv * xhat, axis=0)
        dbeta_c = jnp.sum(dyv, axis=0)

        @pl.when(i == 0)
        def _():
            acc_ref[0, :] = dgamma_c
            acc_ref[1, :] = dbeta_c

        @pl.when(i > 0)
        def _():
            acc_ref[0, :] = acc_ref[0, :] + dgamma_c
            acc_ref[1, :] = acc_ref[1, :] + dbeta_c

        @pl.when(i == K - 1)
        def _():
            pl.semaphore_wait(barrier_sem, 1)
            rdma = pltpu.make_async_remote_copy(
                src_ref=acc_ref,
                dst_ref=comm_ref,
                send_sem=send_sem,
                recv_sem=recv_sem,
                device_id=nbr,
                device_id_type=pl.DeviceIdType.MESH,
            )
            rdma.start()
            rdma.wait()
            out_ref[:, :] = acc_ref[:, :] + comm_ref[:, :]

    return pl.pallas_call(
        body,
        grid=(K,),
        out_shape=jax.ShapeDtypeStruct((2, d), jnp.float32),
        in_specs=[
            pl.BlockSpec((mb, d), lambda i: (i, 0), memory_space=pltpu.VMEM),
            pl.BlockSpec((mb, d), lambda i: (i, 0), memory_space=pltpu.VMEM),
        ],
        out_specs=pl.BlockSpec((2, d), lambda i: (0, 0), memory_space=pltpu.VMEM),
        scratch_shapes=[
            pltpu.VMEM((2, d), jnp.float32),
            pltpu.VMEM((2, d), jnp.float32),
            pltpu.SemaphoreType.DMA,
            pltpu.SemaphoreType.DMA,
        ],
        compiler_params=pltpu.CompilerParams(
            collective_id=0,
            dimension_semantics=("arbitrary",),
        ),
    )(x, dy)
